# baseline (speedup 1.0000x reference)
import functools

import jax
import jax.numpy as jnp
from jax import lax
from jax.experimental import pallas as pl
from jax.experimental.pallas import tpu as pltpu
from jax.experimental.pallas import tpu_sc as plsc

_T = 2048
_S = 2048
_E = 16
_CLS = 65536
_NSH = 8
_NC = 2
_NS = 16
_NW = _NC * _NS
_RPW = _T // _NW
_OUTROWS = _T * 256

_mesh = plsc.VectorSubcoreMesh(core_axis_name="c", subcore_axis_name="s")


@functools.partial(
    pl.kernel,
    mesh=_mesh,
    out_type=jax.ShapeDtypeStruct((_OUTROWS, 128), jnp.float32),
    scratch_types=[
        pltpu.VMEM((512, 128), jnp.float32),
        pltpu.VMEM((128, 128), jnp.int32),
        pltpu.SemaphoreType.DMA,
    ],
)
def _sc_scatter_copy(shifted_hbm, out_hbm, class_v, idx_s, sem):
    wid = lax.axis_index("s") * _NC + lax.axis_index("c")
    m = wid % _NSH
    q = wid // _NSH
    b = (_T - 1 - m) % _NSH
    flo = (_T - 1 - m - b) // _NSH
    pltpu.sync_copy(shifted_hbm.at[b], class_v)

    lanes = lax.iota(jnp.int32, 16)

    def build(rr, carry):
        t = rr // 2
        h = rr % 2
        i = m + _NSH * (q * _RPW + t)
        base = i * 256 + 128 * h
        for c in range(8):
            idx_s[rr, pl.ds(16 * c, 16)] = base + 16 * c + lanes
        return carry

    lax.fori_loop(0, 2 * _RPW, build, 0)

    def body(g, carry):
        descs = []
        for u in range(4):
            rr = g * 4 + u
            t = rr // 2
            h = rr % 2
            a = flo - q * _RPW - t
            d = pltpu.make_async_copy(
                class_v.at[pl.ds(a + 128 * h, 128)],
                out_hbm.at[idx_s.at[rr]],
                sem,
            )
            d.start()
            descs.append(d)
        for d in descs:
            d.wait()
        return carry

    lax.fori_loop(0, (2 * _RPW) // 4, body, 0)


def kernel(table):
    flat = table.reshape(-1)
    padded = jnp.pad(flat, (0, _CLS - flat.shape[0]))
    shifted = jnp.stack(
        [lax.dynamic_slice_in_dim(padded, _E * b, _CLS - 128) for b in range(_NSH)]
    )
    shifted = jnp.pad(shifted, ((0, 0), (0, 128))).reshape(_NSH, 512, 128)
    out = _sc_scatter_copy(shifted)
    return out.reshape(_T, _S, _E)


# speedup vs baseline: 1.0062x; 1.0062x over previous
import functools

import jax
import jax.numpy as jnp
from jax import lax
from jax.experimental import pallas as pl
from jax.experimental.pallas import tpu as pltpu
from jax.experimental.pallas import tpu_sc as plsc

_T = 2048
_S = 2048
_E = 16
_FLAT = (_T + _S - 1) * _E
_ROW = _S * _E

_NC = 2
_NS = 16
_NW = _NC * _NS
_ROWS_PER_W = _T // _NW

_mesh = plsc.VectorSubcoreMesh(core_axis_name="c", subcore_axis_name="s")


@functools.partial(
    pl.kernel,
    mesh=_mesh,
    out_type=jax.ShapeDtypeStruct((_T * _ROW,), jnp.float32),
    scratch_types=[pltpu.VMEM((_FLAT,), jnp.float32)],
)
def _sc_window_copy(table_hbm, out_hbm, table_v):
    wid = lax.axis_index("s") * _NC + lax.axis_index("c")
    pltpu.sync_copy(table_hbm, table_v)
    base = wid * _ROWS_PER_W

    def body(r, carry):
        i = base + r
        src = (_T - 1 - i) * _E
        pltpu.sync_copy(
            table_v.at[pl.ds(src, _ROW)],
            out_hbm.at[pl.ds(i * _ROW, _ROW)],
        )
        return carry

    lax.fori_loop(0, _ROWS_PER_W, body, 0)


def kernel(table):
    flat = table.reshape(-1)
    out = _sc_window_copy(flat)
    return out.reshape(_T, _S, _E)
